# Initial kernel scaffold; baseline (speedup 1.0000x reference)
import functools

import jax
import jax.numpy as jnp
import numpy as np
from jax import lax
from jax.experimental import pallas as pl
from jax.experimental.pallas import tpu as pltpu
from jax.experimental.pallas import tpu_sc as plsc

_NUM_LEVELS = 16
_FPL = 2
_LOG2 = 19
_BASE = 16
_FINEST = 2048
_GROWTH = np.exp((np.log(_FINEST) - np.log(_BASE)) / (_NUM_LEVELS - 1))
_RES = [int(np.floor(_BASE * _GROWTH ** l)) for l in range(_NUM_LEVELS)]
_SIZES = [min(_RES[l] ** 3, 2 ** _LOG2) for l in range(_NUM_LEVELS)]

_B0, _B1 = 4096, 128
_N = _B0 * _B1
_ROWS_PER_BLK = 512

_NC, _NS = 2, 16
_NW = _NC * _NS
_G = _B0 // _NW


def _hash_body(x_ref, y_ref, t_ref, out_ref):
    x = x_ref[...]
    y = y_ref[...]
    t = t_ref[...]
    for l in range(_NUM_LEVELS):
        res = _RES[l]
        gx = jnp.floor(x * res).astype(jnp.int32)
        gy = jnp.floor(y * res).astype(jnp.int32)
        gt = jnp.floor(t * res).astype(jnp.int32)
        h = (gx * 73856093) ^ (gy * 19349663) ^ (gt * 83492791)
        out_ref[l] = jnp.abs(h) % _SIZES[l]


def _hash_indices(coordinates):
    x = coordinates[..., 0]
    y = coordinates[..., 1]
    t = coordinates[..., 2]
    grid = (_B0 // _ROWS_PER_BLK,)
    blk = (_ROWS_PER_BLK, _B1)
    return pl.pallas_call(
        _hash_body,
        grid=grid,
        in_specs=[pl.BlockSpec(blk, lambda i: (i, 0))] * 3,
        out_specs=pl.BlockSpec((_NUM_LEVELS,) + blk, lambda i: (0, i, 0)),
        out_shape=jax.ShapeDtypeStruct((_NUM_LEVELS, _B0, _B1), jnp.int32),
    )(x, y, t)


def _gather_body(idx_hbm, *rest):
    tables = rest[:_NUM_LEVELS]
    out_hbm = rest[_NUM_LEVELS]
    idx_v, rows_v, sem = rest[_NUM_LEVELS + 1:]
    c = lax.axis_index("c")
    s = lax.axis_index("s")
    wid = s * _NC + c
    gbase = wid * _G
    for l in range(_NUM_LEVELS):
        pltpu.sync_copy(idx_hbm.at[l, pl.ds(gbase, _G)], idx_v)
        pltpu.async_copy(tables[l].at[idx_v], rows_v, sem).wait()
        pltpu.sync_copy(rows_v, out_hbm.at[l, pl.ds(gbase, _G)])


_gather_call = functools.partial(
    pl.kernel,
    out_type=jax.ShapeDtypeStruct((_NUM_LEVELS, _B0, _B1, _FPL), jnp.float32),
    mesh=plsc.VectorSubcoreMesh(core_axis_name="c", subcore_axis_name="s"),
    scratch_types=[
        pltpu.VMEM((_G, _B1), jnp.int32),
        pltpu.VMEM((_G, _B1, _FPL), jnp.float32),
        pltpu.SemaphoreType.DMA,
    ],
)(_gather_body)


def kernel(coordinates, tables):
    idx = _hash_indices(coordinates)
    feats = _gather_call(idx, *tables)
    return feats.transpose(1, 2, 0, 3).reshape(_B0, _B1, _NUM_LEVELS * _FPL)


# speedup vs baseline: 3.8256x; 3.8256x over previous
import functools

import jax
import jax.numpy as jnp
import numpy as np
from jax import lax
from jax.experimental import pallas as pl
from jax.experimental.pallas import tpu as pltpu
from jax.experimental.pallas import tpu_sc as plsc

_NUM_LEVELS = 16
_FPL = 2
_LOG2 = 19
_BASE = 16
_FINEST = 2048
_GROWTH = np.exp((np.log(_FINEST) - np.log(_BASE)) / (_NUM_LEVELS - 1))
_RES = [int(np.floor(_BASE * _GROWTH ** l)) for l in range(_NUM_LEVELS)]
_SIZES = [min(_RES[l] ** 3, 2 ** _LOG2) for l in range(_NUM_LEVELS)]

_B0, _B1 = 4096, 128
_N = _B0 * _B1
_ROWS_PER_BLK = 512

_NC, _NS = 2, 16
_NW = _NC * _NS
_CH = _N // _NW
_CHK = 4096


def _hash_body(x_ref, y_ref, t_ref, out_ref):
    x = x_ref[...]
    y = y_ref[...]
    t = t_ref[...]
    for l in range(_NUM_LEVELS):
        res = _RES[l]
        gx = jnp.floor(x * res).astype(jnp.int32)
        gy = jnp.floor(y * res).astype(jnp.int32)
        gt = jnp.floor(t * res).astype(jnp.int32)
        h = (gx * 73856093) ^ (gy * 19349663) ^ (gt * 83492791)
        idx = jnp.abs(h) % _SIZES[l]
        e0 = idx * 2
        out_ref[l, 0] = e0
        out_ref[l, 1] = e0 + 1


def _hash_indices(coordinates):
    x = coordinates[..., 0]
    y = coordinates[..., 1]
    t = coordinates[..., 2]
    grid = (_B0 // _ROWS_PER_BLK,)
    blk = (_ROWS_PER_BLK, _B1)
    return pl.pallas_call(
        _hash_body,
        grid=grid,
        in_specs=[pl.BlockSpec(blk, lambda i: (i, 0))] * 3,
        out_specs=pl.BlockSpec((_NUM_LEVELS, 2) + blk,
                               lambda i: (0, 0, i, 0)),
        out_shape=jax.ShapeDtypeStruct((_NUM_LEVELS, 2, _B0, _B1), jnp.int32),
    )(x, y, t)


def _gather_body(eidx_hbm, *rest):
    tables = rest[:_NUM_LEVELS]
    out_hbm = rest[_NUM_LEVELS]
    idx_v, rows_v, sem = rest[_NUM_LEVELS + 1:]
    c = lax.axis_index("c")
    s = lax.axis_index("s")
    wid = s * _NC + c
    base = wid * _CH
    for l in range(_NUM_LEVELS):
        table = tables[l]
        for p in range(2):

            def chunk(k, _, table=table, l=l, p=p):
                cb = base + k * _CHK
                pltpu.sync_copy(eidx_hbm.at[l, p, pl.ds(cb, _CHK)], idx_v)
                pltpu.async_copy(table.at[idx_v], rows_v, sem).wait()
                pltpu.sync_copy(rows_v, out_hbm.at[l, p, pl.ds(cb, _CHK)])
                return 0

            lax.fori_loop(0, _CH // _CHK, chunk, 0)


_gather_call = functools.partial(
    pl.kernel,
    out_type=jax.ShapeDtypeStruct((_NUM_LEVELS, 2, _N), jnp.float32),
    mesh=plsc.VectorSubcoreMesh(core_axis_name="c", subcore_axis_name="s"),
    scratch_types=[
        pltpu.VMEM((_CHK,), jnp.int32),
        pltpu.VMEM((_CHK,), jnp.float32),
        pltpu.SemaphoreType.DMA,
    ],
    compiler_params=pltpu.CompilerParams(use_tc_tiling_on_sc=False),
)(_gather_body)


def kernel(coordinates, tables):
    eidx = _hash_indices(coordinates).reshape(_NUM_LEVELS, 2, _N)
    flat = [tables[l].reshape(2 * _SIZES[l]) for l in range(_NUM_LEVELS)]
    feats = _gather_call(eidx, *flat)
    return (feats.transpose(2, 0, 1)
            .reshape(_B0, _B1, _NUM_LEVELS * _FPL))
